# baseline (speedup 1.0000x reference)
import functools

import jax
import jax.numpy as jnp
from jax import lax
from jax.experimental import pallas as pl
from jax.experimental.pallas import tpu as pltpu
from jax.experimental.pallas import tpu_sc as plsc

P = 0.15
B, C = 16384, 100
N = B * C

NUM_CORES = 2
NUM_SUBCORES = 16
NW = NUM_CORES * NUM_SUBCORES
LANES = 16

ROWS_W = B // NW
RPR = 128
ROUNDS = ROWS_W // RPR
SUB = RPR * C
GCH = 128
PAD = SUB
COL0 = (0, 16, 32, 48, 64, 80, 84)
MAGIC100 = 41944


def _swap_noise_body(xf_hbm, x2_hbm, rm_hbm, rr_hbm, out_hbm,
                     xv, rm_v, rr_v, idx_l, pos_l, gv, sem, sem_x, sem_o,
                     sem_in):
    c = lax.axis_index("c")
    s = lax.axis_index("s")
    wid = s * NUM_CORES + c
    row_base = wid * ROWS_W
    thresh = jnp.float32(1.0 - P)
    lane = lax.iota(jnp.int32, LANES)
    zeros16 = jnp.zeros((LANES,), jnp.int32)

    def loads(rnd):
        b = rnd % 2
        r0 = row_base + rnd * RPR
        cps = (
            pltpu.make_async_copy(rm_hbm.at[pl.ds(r0, RPR)], rm_v[b], sem_in),
            pltpu.make_async_copy(rr_hbm.at[pl.ds(r0, RPR)], rr_v[b], sem_in),
            pltpu.make_async_copy(
                x2_hbm.at[pl.ds(r0, RPR)], xv.at[pl.ds(0, RPR)], sem_x),
        )
        for cp in cps:
            cp.start()
        return cps

    cps = loads(0)

    def zbody(i, _):
        idx_l[pl.ds(i * LANES, LANES)] = zeros16
        return 0
    lax.fori_loop(0, (SUB + LANES) // LANES, zbody, 0)

    wb = None
    for rnd in range(ROUNDS):
        b = rnd % 2
        r0 = row_base + rnd * RPR
        off = r0 * C
        cps[0].wait()
        cps[1].wait()
        next_cps = None
        if rnd + 1 < ROUNDS:
            r0n = row_base + (rnd + 1) * RPR
            bn = (rnd + 1) % 2
            next_cps = [
                pltpu.make_async_copy(
                    rm_hbm.at[pl.ds(r0n, RPR)], rm_v[bn], sem_in),
                pltpu.make_async_copy(
                    rr_hbm.at[pl.ds(r0n, RPR)], rr_v[bn], sem_in),
                None,
            ]
            next_cps[0].start()
            next_cps[1].start()

        wp = jnp.int32(0)
        for col0 in COL0:
            def cbody(r, wp, col0=col0):
                rm = rm_v[b][r, pl.ds(col0, LANES)]
                rr = rr_v[b][r, pl.ds(col0, LANES)]
                m = rm > thresh
                rowoff = (rr * jnp.float32(B)).astype(jnp.int32)
                k = r * jnp.int32(C) + (col0 + lane)
                idx = (off + k) + rowoff * jnp.int32(C)
                idx = jnp.where(idx >= N, idx - N, idx)
                mi = m.astype(jnp.int32)
                cs = plsc.cumsum(mi)
                dest = jnp.where(m, wp + cs - 1, jnp.int32(PAD))
                plsc.store_scatter(idx_l, [dest], idx)
                plsc.store_scatter(pos_l, [dest], k)
                return wp + plsc.all_reduce_population_count(m)[0]

            wp = plsc.parallel_loop(0, RPR, unroll=8, carry=wp)(cbody)

        pos_l[pl.ds(wp, LANES)] = zeros16 + jnp.int32(PAD)

        n_q = (wp + (GCH - 1)) >> 7

        def gbody(j, _):
            pltpu.async_copy(
                xf_hbm.at[idx_l.at[pl.ds(j * GCH, GCH)]],
                gv.at[pl.ds(j * GCH, GCH)],
                sem,
            )
            return 0

        lax.fori_loop(0, n_q, gbody, 0)

        def dbody(j, _):
            pltpu.make_async_copy(
                xf_hbm.at[pl.ds(0, GCH)], gv.at[pl.ds(0, GCH)], sem).wait()
            return 0

        lax.fori_loop(0, n_q, dbody, 0)
        cps[2].wait()

        def sbody(i, _):
            vals = gv[pl.ds(i * LANES, LANES)]
            pos = pos_l[pl.ds(i * LANES, LANES)]
            prow = (pos * jnp.int32(MAGIC100)) >> 22
            pcol = pos - prow * jnp.int32(C)
            plsc.store_scatter(xv, [prow, pcol], vals)
            return 0

        n_s = (wp + (LANES - 1)) >> 4
        lax.fori_loop(0, n_s, sbody, 0)

        wb = pltpu.make_async_copy(
            xv.at[pl.ds(0, RPR)], out_hbm.at[pl.ds(r0, RPR)], sem_o)
        wb.start()
        if next_cps is not None:
            wb.wait()
            r0n = row_base + (rnd + 1) * RPR
            next_cps[2] = pltpu.make_async_copy(
                x2_hbm.at[pl.ds(r0n, RPR)], xv.at[pl.ds(0, RPR)], sem_x)
            next_cps[2].start()
            cps = next_cps
        else:
            wb.wait()


@functools.partial(
    pl.kernel,
    out_type=jax.ShapeDtypeStruct((B, C), jnp.float32),
    mesh=plsc.VectorSubcoreMesh(
        core_axis_name="c", subcore_axis_name="s",
        num_cores=NUM_CORES, num_subcores=NUM_SUBCORES,
    ),
    scratch_types=[
        pltpu.VMEM((RPR + 1, C), jnp.float32),
        pltpu.VMEM((RPR, C), jnp.float32),
        pltpu.VMEM((RPR, C), jnp.float32),
        pltpu.VMEM((RPR, C), jnp.float32),
        pltpu.VMEM((RPR, C), jnp.float32),
        pltpu.VMEM((SUB + LANES,), jnp.int32),
        pltpu.VMEM((SUB + LANES,), jnp.int32),
        pltpu.VMEM((SUB + LANES,), jnp.float32),
        pltpu.SemaphoreType.DMA,
        pltpu.SemaphoreType.DMA,
        pltpu.SemaphoreType.DMA,
        pltpu.SemaphoreType.DMA,
    ],
    compiler_params=pltpu.CompilerParams(needs_layout_passes=False),
)
def _swap_noise(xf_hbm, x2_hbm, rm_hbm, rr_hbm, out_hbm,
                xv, rm0, rm1, rr0, rr1,
                idx_l, pos_l, gv, sem, sem_x, sem_o, sem_in):
    _swap_noise_body(xf_hbm, x2_hbm, rm_hbm, rr_hbm, out_hbm,
                     xv, (rm0, rm1), (rr0, rr1),
                     idx_l, pos_l, gv, sem, sem_x, sem_o, sem_in)


def kernel(x, rand_mask, rand_rows):
    return _swap_noise(x.reshape(-1), x, rand_mask, rand_rows)


# speedup vs baseline: 1.0440x; 1.0440x over previous
import functools

import jax
import jax.numpy as jnp
from jax import lax
from jax.experimental import pallas as pl
from jax.experimental.pallas import tpu as pltpu
from jax.experimental.pallas import tpu_sc as plsc

P = 0.15
B, C = 16384, 100
N = B * C

NUM_CORES = 2
NUM_SUBCORES = 16
NW = NUM_CORES * NUM_SUBCORES
LANES = 16

ROWS_W = B // NW
RPR = 128
ROUNDS = ROWS_W // RPR
SUB = RPR * C
GCH = 128
PAD = SUB
COL0 = (0, 16, 32, 48, 64, 80, 84)
MAGIC100 = 41944


def _swap_noise_body(xf_hbm, x2_hbm, rm_hbm, rr_hbm, out_hbm,
                     xv, rm_v, rr_v, idx_l, pos_l, gv, sem, sem_x, sem_o,
                     sem_in):
    c = lax.axis_index("c")
    s = lax.axis_index("s")
    wid = s * NUM_CORES + c
    row_base = wid * ROWS_W
    thresh = jnp.float32(1.0 - P)
    lane = lax.iota(jnp.int32, LANES)
    zeros16 = jnp.zeros((LANES,), jnp.int32)

    def loads(rnd):
        b = rnd % 2
        r0 = row_base + rnd * RPR
        cps = (
            pltpu.make_async_copy(rm_hbm.at[pl.ds(r0, RPR)], rm_v[b], sem_in),
            pltpu.make_async_copy(rr_hbm.at[pl.ds(r0, RPR)], rr_v[b], sem_in),
            pltpu.make_async_copy(
                x2_hbm.at[pl.ds(r0, RPR)], xv.at[pl.ds(0, RPR)], sem_x),
        )
        for cp in cps:
            cp.start()
        return cps

    cps = loads(0)

    def zbody(i, _):
        idx_l[pl.ds(i * LANES, LANES)] = zeros16
        return 0
    lax.fori_loop(0, (SUB + LANES) // LANES, zbody, 0)

    wb = None
    for rnd in range(ROUNDS):
        b = rnd % 2
        r0 = row_base + rnd * RPR
        off = r0 * C
        cps[0].wait()
        cps[1].wait()
        next_cps = None
        if rnd + 1 < ROUNDS:
            r0n = row_base + (rnd + 1) * RPR
            bn = (rnd + 1) % 2
            next_cps = [
                pltpu.make_async_copy(
                    rm_hbm.at[pl.ds(r0n, RPR)], rm_v[bn], sem_in),
                pltpu.make_async_copy(
                    rr_hbm.at[pl.ds(r0n, RPR)], rr_v[bn], sem_in),
                None,
            ]
            next_cps[0].start()
            next_cps[1].start()

        wp = jnp.int32(0)
        fired = jnp.int32(0)

        def gbody(j, _):
            pltpu.async_copy(
                xf_hbm.at[idx_l.at[pl.ds(j * GCH, GCH)]],
                gv.at[pl.ds(j * GCH, GCH)],
                sem,
            )
            return 0

        for col0 in COL0:
            def cbody(r, wp, col0=col0):
                rm = rm_v[b][r, pl.ds(col0, LANES)]
                rr = rr_v[b][r, pl.ds(col0, LANES)]
                m = rm > thresh
                rowoff = (rr * jnp.float32(B)).astype(jnp.int32)
                k = r * jnp.int32(C) + (col0 + lane)
                idx = (off + k) + rowoff * jnp.int32(C)
                idx = jnp.where(idx >= N, idx - N, idx)
                mi = m.astype(jnp.int32)
                cs = plsc.cumsum(mi)
                dest = jnp.where(m, wp + cs - 1, jnp.int32(PAD))
                plsc.store_scatter(idx_l, [dest], idx)
                plsc.store_scatter(pos_l, [dest], k)
                return wp + plsc.all_reduce_population_count(m)[0]

            wp = plsc.parallel_loop(0, RPR, unroll=8, carry=wp)(cbody)
            lax.fori_loop(fired, wp >> 7, gbody, 0)
            fired = wp >> 7

        pos_l[pl.ds(wp, LANES)] = zeros16 + jnp.int32(PAD)

        n_q = (wp + (GCH - 1)) >> 7
        lax.fori_loop(fired, n_q, gbody, 0)

        def dbody(j, _):
            pltpu.make_async_copy(
                xf_hbm.at[pl.ds(0, GCH)], gv.at[pl.ds(0, GCH)], sem).wait()
            return 0

        lax.fori_loop(0, n_q, dbody, 0)
        cps[2].wait()

        def sbody(i, _):
            vals = gv[pl.ds(i * LANES, LANES)]
            pos = pos_l[pl.ds(i * LANES, LANES)]
            prow = (pos * jnp.int32(MAGIC100)) >> 22
            pcol = pos - prow * jnp.int32(C)
            plsc.store_scatter(xv, [prow, pcol], vals)
            return 0

        n_s = (wp + (LANES - 1)) >> 4
        lax.fori_loop(0, n_s, sbody, 0)

        wb = pltpu.make_async_copy(
            xv.at[pl.ds(0, RPR)], out_hbm.at[pl.ds(r0, RPR)], sem_o)
        wb.start()
        if next_cps is not None:
            wb.wait()
            r0n = row_base + (rnd + 1) * RPR
            next_cps[2] = pltpu.make_async_copy(
                x2_hbm.at[pl.ds(r0n, RPR)], xv.at[pl.ds(0, RPR)], sem_x)
            next_cps[2].start()
            cps = next_cps
        else:
            wb.wait()


@functools.partial(
    pl.kernel,
    out_type=jax.ShapeDtypeStruct((B, C), jnp.float32),
    mesh=plsc.VectorSubcoreMesh(
        core_axis_name="c", subcore_axis_name="s",
        num_cores=NUM_CORES, num_subcores=NUM_SUBCORES,
    ),
    scratch_types=[
        pltpu.VMEM((RPR + 1, C), jnp.float32),
        pltpu.VMEM((RPR, C), jnp.float32),
        pltpu.VMEM((RPR, C), jnp.float32),
        pltpu.VMEM((RPR, C), jnp.float32),
        pltpu.VMEM((RPR, C), jnp.float32),
        pltpu.VMEM((SUB + LANES,), jnp.int32),
        pltpu.VMEM((SUB + LANES,), jnp.int32),
        pltpu.VMEM((SUB + LANES,), jnp.float32),
        pltpu.SemaphoreType.DMA,
        pltpu.SemaphoreType.DMA,
        pltpu.SemaphoreType.DMA,
        pltpu.SemaphoreType.DMA,
    ],
    compiler_params=pltpu.CompilerParams(needs_layout_passes=False),
)
def _swap_noise(xf_hbm, x2_hbm, rm_hbm, rr_hbm, out_hbm,
                xv, rm0, rm1, rr0, rr1,
                idx_l, pos_l, gv, sem, sem_x, sem_o, sem_in):
    _swap_noise_body(xf_hbm, x2_hbm, rm_hbm, rr_hbm, out_hbm,
                     xv, (rm0, rm1), (rr0, rr1),
                     idx_l, pos_l, gv, sem, sem_x, sem_o, sem_in)


def kernel(x, rand_mask, rand_rows):
    return _swap_noise(x.reshape(-1), x, rand_mask, rand_rows)
